# Initial kernel scaffold; baseline (speedup 1.0000x reference)
import jax
import jax.numpy as jnp
from jax.experimental import pallas as pl


def kernel(inputs, edge_index, edge_ppi, edge_self, W_in, b_in, input_bias, W_ppi1, b_ppi1, W_ppi2, b_ppi2, W_out, b_out):
    raise NotImplementedError("write your pallas kernel here")


# speedup vs baseline: 3.0747x; 3.0747x over previous
import functools

import jax
import jax.numpy as jnp
from jax import lax
from jax.experimental import pallas as pl
from jax.experimental.pallas import tpu as pltpu, tpu_sc as plsc

N = 10000
H = 128
EPS = 1e-12

NC = 2
NS = 16
K = 128

ROW_BLK = 1000


def _h0_body(x_ref, w_ref, b_ref, o_ref):
    y = lax.dot_general(x_ref[...], w_ref[...], (((1,), (1,)), ((), ())),
                        preferred_element_type=jnp.float32)
    y = y + b_ref[...]
    nrm = jnp.sqrt(jnp.sum(y * y, axis=1, keepdims=True))
    o_ref[...] = y / jnp.maximum(nrm, EPS)


def _combine_body(ppi_ref, res_ref, w_ref, b_ref, o_ref):
    y = lax.dot_general(ppi_ref[...], w_ref[...], (((1,), (1,)), ((), ())),
                        preferred_element_type=jnp.float32)
    o_ref[...] = jnp.maximum(y + b_ref[...], 0.0) + res_ref[...]


def _final_body(h_ref, w_ref, b_ref, o_ref):
    y = lax.dot_general(h_ref[...], w_ref[...], (((1,), (1,)), ((), ())),
                        preferred_element_type=jnp.float32)
    o_ref[...] = y + b_ref[...]


def _row_grid(n):
    return (n // ROW_BLK,)


def _tc_h0(x, w, b):
    return pl.pallas_call(
        _h0_body,
        grid=_row_grid(N),
        in_specs=[
            pl.BlockSpec((ROW_BLK, x.shape[1]), lambda i: (i, 0)),
            pl.BlockSpec(w.shape, lambda i: (0, 0)),
            pl.BlockSpec((1, H), lambda i: (0, 0)),
        ],
        out_specs=pl.BlockSpec((ROW_BLK, H), lambda i: (i, 0)),
        out_shape=jax.ShapeDtypeStruct((N, H), jnp.float32),
    )(x, w, b)


def _tc_combine(ppi, res, w, b):
    return pl.pallas_call(
        _combine_body,
        grid=_row_grid(N),
        in_specs=[
            pl.BlockSpec((ROW_BLK, H), lambda i: (i, 0)),
            pl.BlockSpec((ROW_BLK, H), lambda i: (i, 0)),
            pl.BlockSpec((H, H), lambda i: (0, 0)),
            pl.BlockSpec((1, H), lambda i: (0, 0)),
        ],
        out_specs=pl.BlockSpec((ROW_BLK, H), lambda i: (i, 0)),
        out_shape=jax.ShapeDtypeStruct((N, H), jnp.float32),
    )(ppi, res, w, b)


def _tc_final(h, w, b):
    l = w.shape[0]
    return pl.pallas_call(
        _final_body,
        grid=_row_grid(N),
        in_specs=[
            pl.BlockSpec((ROW_BLK, H), lambda i: (i, 0)),
            pl.BlockSpec((l, H), lambda i: (0, 0)),
            pl.BlockSpec((1, l), lambda i: (0, 0)),
        ],
        out_specs=pl.BlockSpec((ROW_BLK, l), lambda i: (i, 0)),
        out_shape=jax.ShapeDtypeStruct((N, l), jnp.float32),
    )(h, w, b)


def _sc_segment_sums(h, src, dst, w2, zeros, e_pad):
    ept = e_pad // NS
    nb = ept // K
    row_stride, row_span = 624, 640

    mesh = plsc.VectorSubcoreMesh(core_axis_name="c", subcore_axis_name="s",
                                  num_cores=NC, num_subcores=NS)

    @functools.partial(
        pl.kernel,
        mesh=mesh,
        out_type=jax.ShapeDtypeStruct((NC, N, H), jnp.float32),
        scratch_types=[
            pltpu.VMEM_SHARED((N, H), jnp.float32),
            pltpu.VMEM((K,), jnp.int32),
            pltpu.VMEM((K,), jnp.int32),
            pltpu.VMEM((K,), jnp.float32),
            pltpu.VMEM((K, H), jnp.float32),
            pltpu.VMEM((K, H), jnp.float32),
            pltpu.SemaphoreType.DMA,
        ],
    )
    def sc_kernel(h_hbm, src_hbm, dst_hbm, w2_hbm, z_hbm, out_hbm,
                  acc, src_v, dst_v, w_v, rows_v, prod_v, sem):
        c = lax.axis_index("c")
        s = lax.axis_index("s")

        pltpu.sync_copy(z_hbm.at[pl.ds(0, K)], rows_v)
        for z in range(row_span // K):
            pltpu.sync_copy(rows_v,
                            acc.at[pl.ds(s * row_stride + z * K, K)])
        plsc.subcore_barrier()

        def batch_body(b, carry):
            base = s * ept + b * K
            pltpu.sync_copy(src_hbm.at[pl.ds(base, K)], src_v)
            pltpu.sync_copy(dst_hbm.at[pl.ds(base, K)], dst_v)
            pltpu.sync_copy(w2_hbm.at[c, pl.ds(base, K)], w_v)
            pltpu.async_copy(h_hbm.at[src_v], rows_v, sem).wait()

            def group_body(g, carry2):
                w16 = w_v[pl.ds(g * 16, 16)]
                for j in range(16):
                    e = g * 16 + j
                    wb = w16[j]
                    for ch in range(H // 16):
                        sl = pl.ds(ch * 16, 16)
                        prod_v[e, sl] = rows_v[e, sl] * wb
                return carry2

            lax.fori_loop(0, K // 16, group_body, 0)
            pltpu.sync_copy(prod_v, acc.at[dst_v], add=True)
            return carry

        lax.fori_loop(0, nb, batch_body, 0)
        plsc.subcore_barrier()

        pltpu.sync_copy(acc.at[pl.ds(s * row_stride, row_span)],
                        out_hbm.at[c, pl.ds(s * row_stride, row_span)])

    return sc_kernel(h, src, dst, w2, zeros)


def kernel(inputs, edge_index, edge_ppi, edge_self, W_in, b_in, input_bias,
           W_ppi1, b_ppi1, W_ppi2, b_ppi2, W_out, b_out):
    e = edge_index.shape[1]
    e_pad = ((e + NS * K - 1) // (NS * K)) * (NS * K)
    pad = e_pad - e

    src = jnp.concatenate([edge_index[0], jnp.zeros((pad,), jnp.int32)])
    dst = jnp.concatenate([edge_index[1], jnp.zeros((pad,), jnp.int32)])
    wpad = jnp.zeros((pad,), jnp.float32)
    w2 = jnp.stack([jnp.concatenate([edge_self, wpad]),
                    jnp.concatenate([edge_ppi, wpad])])
    zeros = jnp.zeros((K, H), jnp.float32)

    bias0 = (b_in + input_bias).reshape(1, H)
    h = _tc_h0(inputs, W_in, bias0)

    for w, b in ((W_ppi1, b_ppi1), (W_ppi2, b_ppi2)):
        sums = _sc_segment_sums(h, src, dst, w2, zeros, e_pad)
        h = _tc_combine(sums[1], sums[0], w, b.reshape(1, H))

    return _tc_final(h, W_out, b_out.reshape(1, W_out.shape[0]))
